# baseline (speedup 1.0000x reference)
import jax
import jax.numpy as jnp
from jax import lax
from jax.experimental import pallas as pl
from jax.experimental.pallas import tpu as pltpu
from jax.experimental.pallas import tpu_sc as plsc

N_OUT = 50000
N_IN = 12500
B = 4
C = 128
NW = 32
R = 64
NCHUNK = (B * N_OUT) // R
CH_MAX = -(-NCHUNK // NW)
FULL_W = NCHUNK - (CH_MAX - 1) * NW
LANES = 16
GROUPS = R // LANES
CSL = C // LANES


def _body(x_hbm, idx_hbm, val_hbm, out_hbm,
          idxs_v, vals_v, g_v, o_v, gsem0, gsem1):
    cid = lax.axis_index("c")
    sid = lax.axis_index("s")
    wid = sid * 2 + cid
    nch = jnp.where(wid < FULL_W, CH_MAX, CH_MAX - 1)
    gsems = (gsem0, gsem1)

    pltpu.sync_copy(idx_hbm.at[wid], idxs_v)
    pltpu.sync_copy(val_hbm.at[wid], vals_v)

    def start_gather(t, buf):
        for j in range(3):
            pltpu.async_copy(
                x_hbm.at[idxs_v.at[pl.ds(t * 3 * R + j * R, R)]],
                g_v.at[buf, pl.ds(j * R, R)],
                gsems[buf],
            )

    def wait_gather(t, buf):
        for j in range(3):
            pltpu.make_async_copy(
                x_hbm.at[idxs_v.at[pl.ds(t * 3 * R + j * R, R)]],
                g_v.at[buf, pl.ds(j * R, R)],
                gsems[buf],
            ).wait()

    def compute_write(t, buf):
        def group(q, _):
            wv = [
                vals_v[pl.ds(t * 3 * R + j * R + q * LANES, LANES)]
                for j in range(3)
            ]
            for k in range(LANES):
                i = q * LANES + k
                w0, w1, w2 = wv[0][k], wv[1][k], wv[2][k]
                for c in range(CSL):
                    sl = pl.ds(c * LANES, LANES)
                    o_v[buf, i, sl] = (
                        g_v[buf, i, sl] * w0
                        + g_v[buf, R + i, sl] * w1
                        + g_v[buf, 2 * R + i, sl] * w2
                    )
            return _

        lax.fori_loop(0, GROUPS, group, 0)
        base = (wid + t * NW) * R
        pltpu.sync_copy(o_v.at[buf], out_hbm.at[pl.ds(base, R)])

    start_gather(0, 0)

    def pair(p, _):
        for b in range(2):
            t = 2 * p + b
            tn = t + 1

            @pl.when(tn < nch)
            def _prefetch():
                start_gather(tn, 1 - b)

            @pl.when(t < nch)
            def _do():
                wait_gather(t, b)
                compute_write(t, b)
        return _

    lax.fori_loop(0, CH_MAX // 2, pair, 0)


@jax.jit
def _run(x2, idx_resh, val_resh):
    mesh = plsc.VectorSubcoreMesh(core_axis_name="c", subcore_axis_name="s")
    f = pl.kernel(
        _body,
        out_type=jax.ShapeDtypeStruct((B * N_OUT, C), jnp.float32),
        mesh=mesh,
        scratch_types=[
            pltpu.VMEM((CH_MAX * 3 * R,), jnp.int32),
            pltpu.VMEM((CH_MAX * 3 * R,), jnp.float32),
            pltpu.VMEM((2, 3 * R, C), jnp.float32),
            pltpu.VMEM((2, R, C), jnp.float32),
            pltpu.SemaphoreType.DMA,
            pltpu.SemaphoreType.DMA,
        ],
    )
    return f(x2, idx_resh, val_resh)


def kernel(x, index, value):
    col = index[1]
    idx_sl = (
        col.reshape(N_OUT, 3).T.reshape(3, 1, N_OUT)
        + (jnp.arange(B, dtype=jnp.int32) * N_IN).reshape(1, B, 1)
    ).reshape(3, B * N_OUT)
    val_sl = jnp.broadcast_to(
        value.reshape(N_OUT, 3).T.reshape(3, 1, N_OUT), (3, B, N_OUT)
    ).reshape(3, B * N_OUT)
    pad = CH_MAX * NW - NCHUNK

    def layout(a):
        a = jnp.concatenate([a, jnp.zeros((3, pad * R), a.dtype)], axis=1)
        return a.reshape(3, CH_MAX, NW, R).transpose(2, 1, 0, 3).reshape(
            NW, CH_MAX * 3 * R
        )

    x2 = x.reshape(B * N_IN, C)
    out2 = _run(x2, layout(idx_sl), layout(val_sl))
    return out2.reshape(B, N_OUT, C)
